# baseline (speedup 1.0000x reference)
import functools

import jax
import jax.numpy as jnp
from jax import lax
from jax.experimental import pallas as pl
from jax.experimental.pallas import tpu as pltpu
from jax.experimental.pallas import tpu_sc as plsc

N_ITEMS = 100000
ORDER = 64
PERIOD = 365.25
BATCH = 16384

NC = 2
NS = 16
L = 16
NW = NC * NS
B_PER_W = BATCH // NW
IDX_CHUNK = 128
N_CHUNKS = B_PER_W // IDX_CHUNK
N_GROUPS = B_PER_W // L

TWO_PI_OVER_P = float(2.0 * 3.141592653589793 / PERIOD)
TWO_OVER_PI = float(2.0 / 3.141592653589793)
PIO2_HI = 1.57079637050628662109375
PIO2_LO = 4.37113900018624283e-8
MAGIC = 12582912.0


def _cos_sin(theta):
    kf = (theta * TWO_OVER_PI + MAGIC) - MAGIC
    r = (theta - kf * PIO2_HI) + kf * PIO2_LO
    q = kf.astype(jnp.int32) & 3
    r2 = r * r
    sp = r + r * r2 * (-1.6666654611e-1
                       + r2 * (8.3321608736e-3 + r2 * (-1.9515295891e-4)))
    cp = 1.0 - 0.5 * r2 + r2 * r2 * (4.166664568298827e-2
                                     + r2 * (-1.388731625493765e-3
                                             + r2 * 2.443315711809948e-5))
    odd = (q & 1) == 1
    s1 = jnp.where(odd, cp, sp)
    c1 = jnp.where(odd, -sp, cp)
    neg = q >= 2
    s1 = jnp.where(neg, -s1, s1)
    c1 = jnp.where(neg, -c1, c1)
    return c1, s1


def _sc_kernel(t_hbm, idx_hbm, a_hbm, b_hbm, out_hbm,
               idx_v, t_v, a_rows, b_rows, out_v, sem):
    wid = lax.axis_index("s") * NC + lax.axis_index("c")

    pltpu.sync_copy(idx_hbm.at[wid], idx_v)
    pltpu.sync_copy(t_hbm.at[wid], t_v)

    copies = []
    for j in range(N_CHUNKS):
        sl = pl.ds(j * IDX_CHUNK, IDX_CHUNK)
        copies.append(pltpu.async_copy(a_hbm.at[idx_v.at[j]], a_rows.at[sl], sem))
        copies.append(pltpu.async_copy(b_hbm.at[idx_v.at[j]], b_rows.at[sl], sem))
    for c in copies:
        c.wait()

    def group_body(g, carry):
        tv = t_v[pl.ds(g * L, L)]
        c1, s1 = _cos_sin(tv * TWO_PI_OVER_P)
        row = g * L + lax.iota(jnp.int32, L)

        def harmonic(n, st):
            cn, sn, acc = st
            col = jnp.full((L,), n, jnp.int32)
            av = plsc.load_gather(a_rows, [row, col])
            bv = plsc.load_gather(b_rows, [row, col])
            acc = acc + cn * av + sn * bv
            return (cn * c1 - sn * s1, sn * c1 + cn * s1, acc)

        acc0 = jnp.zeros((L,), jnp.float32)
        _, _, acc = lax.fori_loop(0, ORDER, harmonic, (c1, s1, acc0))
        out_v[pl.ds(g * L, L)] = acc
        return carry

    lax.fori_loop(0, N_GROUPS, group_body, 0)
    pltpu.sync_copy(out_v, out_hbm.at[wid])


def kernel(t, item_id, a_table, b_table):
    t2 = t.reshape(NW, B_PER_W)
    idx2 = item_id.astype(jnp.int32).reshape(NW, N_CHUNKS, IDX_CHUNK)

    mesh = plsc.VectorSubcoreMesh(core_axis_name="c", subcore_axis_name="s")
    run = pl.kernel(
        _sc_kernel,
        mesh=mesh,
        compiler_params=pltpu.CompilerParams(
            needs_layout_passes=False, use_tc_tiling_on_sc=False),
        out_type=jax.ShapeDtypeStruct((NW, B_PER_W), jnp.float32),
        scratch_types=[
            pltpu.VMEM((N_CHUNKS, IDX_CHUNK), jnp.int32),
            pltpu.VMEM((B_PER_W,), jnp.float32),
            pltpu.VMEM((B_PER_W, ORDER), jnp.float32),
            pltpu.VMEM((B_PER_W, ORDER), jnp.float32),
            pltpu.VMEM((B_PER_W,), jnp.float32),
            pltpu.SemaphoreType.DMA,
        ],
    )
    out = run(t2, idx2, a_table, b_table)
    return out.reshape(BATCH, 1)
